# baseline (speedup 1.0000x reference)
import functools

import jax
import jax.numpy as jnp
from jax import lax
from jax.experimental import pallas as pl
from jax.experimental.pallas import tpu as pltpu
from jax.experimental.pallas import tpu_sc as plsc

EMB = 64
NC = 2
NS = 16
NW = NC * NS
CHUNK = 512


def _sc_gather_pair(table, idx_a, idx_b):
    n = idx_a.shape[0]
    per_w = n // NW
    n_chunks = per_w // CHUNK
    assert per_w % CHUNK == 0 and n % NW == 0

    mesh = plsc.VectorSubcoreMesh(core_axis_name="c", subcore_axis_name="s")

    @functools.partial(
        pl.kernel,
        out_type=(
            jax.ShapeDtypeStruct((n, EMB), jnp.float32),
            jax.ShapeDtypeStruct((n, EMB), jnp.float32),
        ),
        mesh=mesh,
        scratch_types=[
            pltpu.VMEM((per_w,), jnp.int32),
            pltpu.VMEM((per_w,), jnp.int32),
            pltpu.VMEM((CHUNK, EMB), jnp.float32),
            pltpu.VMEM((CHUNK, EMB), jnp.float32),
            pltpu.SemaphoreType.DMA,
            pltpu.SemaphoreType.DMA,
        ],
        compiler_params=pltpu.CompilerParams(use_tc_tiling_on_sc=False),
    )
    def gather_kernel(idx_a_hbm, idx_b_hbm, table_hbm, out_a_hbm, out_b_hbm,
                      idx_a_v, idx_b_v, buf0, buf1, sem0, sem1):
        wid = lax.axis_index("s") * NC + lax.axis_index("c")
        base = wid * per_w
        pltpu.sync_copy(idx_a_hbm.at[pl.ds(base, per_w)], idx_a_v)
        pltpu.sync_copy(idx_b_hbm.at[pl.ds(base, per_w)], idx_b_v)

        def one_array(idx_v, out_hbm):
            def body(c, _):
                off = c * CHUNK
                pltpu.async_copy(
                    table_hbm.at[idx_v.at[pl.ds(off, CHUNK)]], buf0, sem0
                ).wait()
                pltpu.sync_copy(buf0, out_hbm.at[pl.ds(base + off, CHUNK)])
                return ()

            lax.fori_loop(0, n_chunks, body, (), unroll=False)

        one_array(idx_a_v, out_a_hbm)
        one_array(idx_b_v, out_b_hbm)

    return gather_kernel(idx_a, idx_b, table)


def _tc_proj_pair(emb_a, emb_b, Wt, b_row):
    n = emb_a.shape[0]
    BLK = 8192
    grid = n // BLK

    def body(xa_ref, xb_ref, wt_ref, b_ref, oa_ref, ob_ref):
        wt = wt_ref[...]
        bb = b_ref[...]
        xa = jnp.maximum(xa_ref[...], 0.0)
        xb = jnp.maximum(xb_ref[...], 0.0)
        oa_ref[...] = jnp.dot(xa, wt, preferred_element_type=jnp.float32) + bb
        ob_ref[...] = jnp.dot(xb, wt, preferred_element_type=jnp.float32) + bb

    return pl.pallas_call(
        body,
        grid=(grid,),
        in_specs=[
            pl.BlockSpec((BLK, EMB), lambda i: (i, 0)),
            pl.BlockSpec((BLK, EMB), lambda i: (i, 0)),
            pl.BlockSpec((EMB, EMB), lambda i: (0, 0)),
            pl.BlockSpec((1, EMB), lambda i: (0, 0)),
        ],
        out_specs=[
            pl.BlockSpec((BLK, EMB), lambda i: (i, 0)),
            pl.BlockSpec((BLK, EMB), lambda i: (i, 0)),
        ],
        out_shape=(
            jax.ShapeDtypeStruct((n, EMB), jnp.float32),
            jax.ShapeDtypeStruct((n, EMB), jnp.float32),
        ),
    )(emb_a, emb_b, Wt, b_row)


def kernel(tensor_day, tensor_diagnoses, table, W, b):
    B, L = tensor_day.shape
    n = B * L
    idx_diag = tensor_diagnoses.reshape(n).astype(jnp.int32)
    idx_day = tensor_day.reshape(n).astype(jnp.int32)

    emb_diag_flat, emb_day_flat = _sc_gather_pair(table, idx_diag, idx_day)

    proj_diag_flat, proj_day_flat = _tc_proj_pair(
        emb_diag_flat, emb_day_flat, W.T, b.reshape(1, EMB)
    )

    emb_diags = emb_diag_flat.reshape(B, L, EMB)
    emb_day = emb_day_flat.reshape(B, L, EMB)
    proj_diags = proj_diag_flat.reshape(B, L, EMB)
    proj_day = proj_day_flat.reshape(B, L, EMB)

    return (
        (emb_diags, emb_day),
        (proj_diags, proj_day),
        (emb_diags, emb_day),
        (proj_diags, proj_day),
    )


# speedup vs baseline: 1.2539x; 1.2539x over previous
import functools

import jax
import jax.numpy as jnp
from jax import lax
from jax.experimental import pallas as pl
from jax.experimental.pallas import tpu as pltpu
from jax.experimental.pallas import tpu_sc as plsc

EMB = 64
NC = 2
NS = 16
NW = NC * NS
CHUNK = 512


def _sc_gather_pair(table, idx_a, idx_b):
    n = idx_a.shape[0]
    per_w = n // NW
    n_chunks = per_w // CHUNK
    assert per_w % CHUNK == 0 and n % NW == 0

    mesh = plsc.VectorSubcoreMesh(core_axis_name="c", subcore_axis_name="s")

    @functools.partial(
        pl.kernel,
        out_type=(
            jax.ShapeDtypeStruct((n, EMB), jnp.float32),
            jax.ShapeDtypeStruct((n, EMB), jnp.float32),
        ),
        mesh=mesh,
        scratch_types=[
            pltpu.VMEM((per_w,), jnp.int32),
            pltpu.VMEM((per_w,), jnp.int32),
            pltpu.VMEM((CHUNK, EMB), jnp.float32),
            pltpu.VMEM((CHUNK, EMB), jnp.float32),
            pltpu.SemaphoreType.DMA,
            pltpu.SemaphoreType.DMA,
        ],
        compiler_params=pltpu.CompilerParams(use_tc_tiling_on_sc=False),
    )
    def gather_kernel(idx_a_hbm, idx_b_hbm, table_hbm, out_a_hbm, out_b_hbm,
                      idx_a_v, idx_b_v, buf0, buf1, sem0, sem1):
        wid = lax.axis_index("s") * NC + lax.axis_index("c")
        base = wid * per_w
        pltpu.sync_copy(idx_a_hbm.at[pl.ds(base, per_w)], idx_a_v)
        pltpu.sync_copy(idx_b_hbm.at[pl.ds(base, per_w)], idx_b_v)

        def one_array(idx_v, out_hbm):
            def body(c, _):
                off = c * CHUNK
                pltpu.async_copy(
                    table_hbm.at[idx_v.at[pl.ds(off, CHUNK)]], buf0, sem0
                ).wait()
                pltpu.sync_copy(buf0, out_hbm.at[pl.ds(base + off, CHUNK)])
                return ()

            lax.fori_loop(0, n_chunks, body, (), unroll=False)

        one_array(idx_a_v, out_a_hbm)
        one_array(idx_b_v, out_b_hbm)

    return gather_kernel(idx_a, idx_b, table)


def _tc_finalize(tmp_a, tmp_b, W, b, L, B):
    BB = 1024
    nb = B // BB

    def body(ta_ref, tb_ref, w_ref, b_ref,
             ea1_ref, ea2_ref, eb1_ref, eb2_ref,
             pa1_ref, pa2_ref, pb1_ref, pb2_ref):
        w = w_ref[...]
        bc = b_ref[...]
        xa = ta_ref[0]
        xb = tb_ref[0]
        ea = xa.T
        eb = xb.T
        ea1_ref[0] = ea
        ea2_ref[0] = ea
        eb1_ref[0] = eb
        eb2_ref[0] = eb
        pa = jnp.dot(w, jnp.maximum(ea, 0.0),
                     preferred_element_type=jnp.float32) + bc
        pb = jnp.dot(w, jnp.maximum(eb, 0.0),
                     preferred_element_type=jnp.float32) + bc
        pa1_ref[0] = pa
        pa2_ref[0] = pa
        pb1_ref[0] = pb
        pb2_ref[0] = pb

    in_blk = pl.BlockSpec((1, BB, EMB), lambda l, j: (l, j, 0))
    out_blk = pl.BlockSpec((1, EMB, BB), lambda l, j: (l, 0, j))
    out_sh = jax.ShapeDtypeStruct((L, EMB, B), jnp.float32)
    return pl.pallas_call(
        body,
        grid=(L, nb),
        in_specs=[
            in_blk,
            in_blk,
            pl.BlockSpec((EMB, EMB), lambda l, j: (0, 0)),
            pl.BlockSpec((EMB, 1), lambda l, j: (0, 0)),
        ],
        out_specs=[out_blk] * 8,
        out_shape=(out_sh,) * 8,
    )(tmp_a, tmp_b, W, b)


def kernel(tensor_day, tensor_diagnoses, table, W, b):
    B, L = tensor_day.shape
    n = B * L
    idx_diag = tensor_diagnoses.T.reshape(n).astype(jnp.int32)
    idx_day = tensor_day.T.reshape(n).astype(jnp.int32)

    tmp_diag, tmp_day = _sc_gather_pair(table, idx_diag, idx_day)
    tmp_diag = tmp_diag.reshape(L, B, EMB)
    tmp_day = tmp_day.reshape(L, B, EMB)

    (ed_x, ed_y, ey_x, ey_y, pd_x, pd_y, py_x, py_y) = _tc_finalize(
        tmp_diag, tmp_day, W, b.reshape(EMB, 1), L, B
    )

    def to_logical(x):
        return jnp.transpose(x, (2, 0, 1))

    return (
        (to_logical(ed_x), to_logical(ey_x)),
        (to_logical(pd_x), to_logical(py_x)),
        (to_logical(ed_y), to_logical(ey_y)),
        (to_logical(pd_y), to_logical(py_y)),
    )


# speedup vs baseline: 1.3550x; 1.0806x over previous
import functools

import jax
import jax.numpy as jnp
from jax import lax
from jax.experimental import pallas as pl
from jax.experimental.pallas import tpu as pltpu
from jax.experimental.pallas import tpu_sc as plsc

EMB = 64
NC = 2
NS = 16
NW = NC * NS
CHUNK = 512


def _sc_gather_pair(table, idx_a, idx_b):
    n = idx_a.shape[0]
    per_w = n // NW
    n_chunks = per_w // CHUNK
    assert per_w % CHUNK == 0 and n % NW == 0

    mesh = plsc.VectorSubcoreMesh(core_axis_name="c", subcore_axis_name="s")

    @functools.partial(
        pl.kernel,
        out_type=(
            jax.ShapeDtypeStruct((n, EMB), jnp.float32),
            jax.ShapeDtypeStruct((n, EMB), jnp.float32),
        ),
        mesh=mesh,
        scratch_types=[
            pltpu.VMEM((per_w,), jnp.int32),
            pltpu.VMEM((per_w,), jnp.int32),
            pltpu.VMEM((CHUNK, EMB), jnp.float32),
            pltpu.VMEM((CHUNK, EMB), jnp.float32),
            pltpu.SemaphoreType.DMA,
            pltpu.SemaphoreType.DMA,
        ],
        compiler_params=pltpu.CompilerParams(use_tc_tiling_on_sc=False),
    )
    def gather_kernel(idx_a_hbm, idx_b_hbm, table_hbm, out_a_hbm, out_b_hbm,
                      idx_a_v, idx_b_v, buf0, buf1, sem0, sem1):
        wid = lax.axis_index("s") * NC + lax.axis_index("c")
        base = wid * per_w
        pltpu.sync_copy(idx_a_hbm.at[pl.ds(base, per_w)], idx_a_v)
        pltpu.sync_copy(idx_b_hbm.at[pl.ds(base, per_w)], idx_b_v)

        def one_array(idx_v, out_hbm):
            def body(c, _):
                off = c * CHUNK
                pltpu.async_copy(
                    table_hbm.at[idx_v.at[pl.ds(off, CHUNK)]], buf0, sem0
                ).wait()
                pltpu.sync_copy(buf0, out_hbm.at[pl.ds(base + off, CHUNK)])
                return ()

            lax.fori_loop(0, n_chunks, body, (), unroll=False)

        one_array(idx_a_v, out_a_hbm)
        one_array(idx_b_v, out_b_hbm)

    return gather_kernel(idx_a, idx_b, table)


def _tc_finalize(tmp_a, tmp_b, W, b, L, B):
    nb = B // 1024

    def body(ta_ref, tb_ref, w_ref, b_ref,
             ea1_ref, ea2_ref, eb1_ref, eb2_ref,
             pa1_ref, pa2_ref, pb1_ref, pb2_ref):
        w = w_ref[...]
        bc = b_ref[...]

        def one(t_ref, e1_ref, e2_ref, p1_ref, p2_ref):
            xt = t_ref[0, 0].T
            e = jnp.concatenate([xt[:EMB], xt[EMB:]], axis=1)
            e1_ref[0] = e
            e2_ref[0] = e
            p = jnp.dot(w, jnp.maximum(e, 0.0),
                        preferred_element_type=jnp.float32) + bc
            p1_ref[0] = p
            p2_ref[0] = p

        one(ta_ref, ea1_ref, ea2_ref, pa1_ref, pa2_ref)
        one(tb_ref, eb1_ref, eb2_ref, pb1_ref, pb2_ref)

    in_blk = pl.BlockSpec((1, 1, 512, 128), lambda l, j: (l, j, 0, 0))
    out_blk = pl.BlockSpec((1, EMB, 1024), lambda l, j: (l, 0, j))
    out_sh = jax.ShapeDtypeStruct((L, EMB, B), jnp.float32)
    return pl.pallas_call(
        body,
        grid=(L, nb),
        in_specs=[
            in_blk,
            in_blk,
            pl.BlockSpec((EMB, EMB), lambda l, j: (0, 0)),
            pl.BlockSpec((EMB, 1), lambda l, j: (0, 0)),
        ],
        out_specs=[out_blk] * 8,
        out_shape=(out_sh,) * 8,
    )(tmp_a, tmp_b, W, b)


def _permuted_flat_indices(t, L, B):
    tt = t.T.reshape(L, B // 1024, 2, 512)
    return tt.transpose(0, 1, 3, 2).reshape(L * B).astype(jnp.int32)


def kernel(tensor_day, tensor_diagnoses, table, W, b):
    B, L = tensor_day.shape
    n = B * L
    idx_diag = _permuted_flat_indices(tensor_diagnoses, L, B)
    idx_day = _permuted_flat_indices(tensor_day, L, B)

    tmp_diag, tmp_day = _sc_gather_pair(table, idx_diag, idx_day)
    tmp_diag = tmp_diag.reshape(L, B // 1024, 512, 128)
    tmp_day = tmp_day.reshape(L, B // 1024, 512, 128)

    (ed_x, ed_y, ey_x, ey_y, pd_x, pd_y, py_x, py_y) = _tc_finalize(
        tmp_diag, tmp_day, W, b.reshape(EMB, 1), L, B
    )

    def to_logical(x):
        return jnp.transpose(x, (2, 0, 1))

    return (
        (to_logical(ed_x), to_logical(ey_x)),
        (to_logical(pd_x), to_logical(py_x)),
        (to_logical(ed_y), to_logical(ey_y)),
        (to_logical(pd_y), to_logical(py_y)),
    )


# speedup vs baseline: 1.9620x; 1.4480x over previous
import functools

import jax
import jax.numpy as jnp
from jax import lax
from jax.experimental import pallas as pl
from jax.experimental.pallas import tpu as pltpu
from jax.experimental.pallas import tpu_sc as plsc

EMB = 64
NC = 2
NS = 16
NW = NC * NS
CHUNK = 512


def _sc_gather_pair(table, idx_a, idx_b):
    n = idx_a.shape[0]
    per_w = n // NW
    n_chunks = per_w // CHUNK
    assert per_w % CHUNK == 0 and n % NW == 0

    mesh = plsc.VectorSubcoreMesh(core_axis_name="c", subcore_axis_name="s")

    @functools.partial(
        pl.kernel,
        out_type=(
            jax.ShapeDtypeStruct((n, EMB), jnp.float32),
            jax.ShapeDtypeStruct((n, EMB), jnp.float32),
        ),
        mesh=mesh,
        scratch_types=[
            pltpu.VMEM((per_w,), jnp.int32),
            pltpu.VMEM((per_w,), jnp.int32),
            pltpu.VMEM((CHUNK, EMB), jnp.float32),
            pltpu.VMEM((CHUNK, EMB), jnp.float32),
            pltpu.SemaphoreType.DMA,
            pltpu.SemaphoreType.DMA,
        ],
        compiler_params=pltpu.CompilerParams(use_tc_tiling_on_sc=False),
    )
    def gather_kernel(idx_a_hbm, idx_b_hbm, table_hbm, out_a_hbm, out_b_hbm,
                      idx_a_v, idx_b_v, buf0, buf1, sem0, sem1):
        wid = lax.axis_index("s") * NC + lax.axis_index("c")
        base = wid * per_w
        pltpu.sync_copy(idx_a_hbm.at[pl.ds(base, per_w)], idx_a_v)
        pltpu.sync_copy(idx_b_hbm.at[pl.ds(base, per_w)], idx_b_v)

        def one_array(idx_v, out_hbm):
            def body(c, _):
                off = c * CHUNK
                pltpu.async_copy(
                    table_hbm.at[idx_v.at[pl.ds(off, CHUNK)]], buf0, sem0
                ).wait()
                pltpu.sync_copy(buf0, out_hbm.at[pl.ds(base + off, CHUNK)])
                return ()

            lax.fori_loop(0, n_chunks, body, (), unroll=False)

        one_array(idx_a_v, out_a_hbm)
        one_array(idx_b_v, out_b_hbm)

    return gather_kernel(idx_a, idx_b, table)


def _tc_transpose_table(table):
    V = table.shape[0]
    VB = 4096
    K = VB // 2
    grid = (V + VB - 1) // VB

    def body(x_ref, y_ref):
        xt = x_ref[...].T
        y_ref[...] = jnp.concatenate([xt[:K], xt[K:]], axis=1)

    return pl.pallas_call(
        body,
        grid=(grid,),
        in_specs=[pl.BlockSpec((EMB, VB), lambda j: (0, j))],
        out_specs=pl.BlockSpec((K, 2 * EMB), lambda j: (j, 0)),
        out_shape=jax.ShapeDtypeStruct((grid * K, 2 * EMB), jnp.float32),
    )(table.T)


def _tc_finalize(tmp_a, tmp_b, W, b, L, B):
    nb = B // 1024

    def body(ta_ref, tb_ref, w_ref, b_ref,
             ea1_ref, ea2_ref, eb1_ref, eb2_ref,
             pa1_ref, pa2_ref, pb1_ref, pb2_ref):
        w = w_ref[...]
        bc = b_ref[...]

        def one(t_ref, e1_ref, e2_ref, p1_ref, p2_ref):
            xt = t_ref[0, 0].T
            e = jnp.concatenate([xt[:EMB], xt[EMB:]], axis=1)
            e1_ref[0] = e
            e2_ref[0] = e
            p = jnp.dot(w, jnp.maximum(e, 0.0),
                        preferred_element_type=jnp.float32) + bc
            p1_ref[0] = p
            p2_ref[0] = p

        one(ta_ref, ea1_ref, ea2_ref, pa1_ref, pa2_ref)
        one(tb_ref, eb1_ref, eb2_ref, pb1_ref, pb2_ref)

    in_blk = pl.BlockSpec((1, 1, 512, 128), lambda l, j: (l, j, 0, 0))
    out_blk = pl.BlockSpec((1, EMB, 1024), lambda l, j: (l, 0, j))
    out_sh = jax.ShapeDtypeStruct((L, EMB, B), jnp.float32)
    return pl.pallas_call(
        body,
        grid=(L, nb),
        in_specs=[
            in_blk,
            in_blk,
            pl.BlockSpec((EMB, EMB), lambda l, j: (0, 0)),
            pl.BlockSpec((EMB, 1), lambda l, j: (0, 0)),
        ],
        out_specs=[out_blk] * 8,
        out_shape=(out_sh,) * 8,
    )(tmp_a, tmp_b, W, b)


def _permuted_flat_indices(t, L, B):
    v = t.astype(jnp.int32)
    r = v & (4096 - 1)
    i = (v & ~(4096 - 1)) + ((r & (2048 - 1)) << 1) + (r >> 11)
    tt = i.T.reshape(L, B // 1024, 2, 512)
    return tt.transpose(0, 1, 3, 2).reshape(L * B).astype(jnp.int32)


def kernel(tensor_day, tensor_diagnoses, table, W, b):
    B, L = tensor_day.shape
    n = B * L
    idx_diag = _permuted_flat_indices(tensor_diagnoses, L, B)
    idx_day = _permuted_flat_indices(tensor_day, L, B)

    table2 = _tc_transpose_table(table)
    table_lin = table2.reshape(table2.shape[0] * 2, EMB)

    tmp_diag, tmp_day = _sc_gather_pair(table_lin, idx_diag, idx_day)
    tmp_diag = tmp_diag.reshape(L, B // 1024, 512, 128)
    tmp_day = tmp_day.reshape(L, B // 1024, 512, 128)

    (ed_x, ed_y, ey_x, ey_y, pd_x, pd_y, py_x, py_y) = _tc_finalize(
        tmp_diag, tmp_day, W, b.reshape(EMB, 1), L, B
    )

    def to_logical(x):
        return jnp.transpose(x, (2, 0, 1))

    return (
        (to_logical(ed_x), to_logical(ey_x)),
        (to_logical(pd_x), to_logical(py_x)),
        (to_logical(ed_y), to_logical(ey_y)),
        (to_logical(pd_y), to_logical(py_y)),
    )


# speedup vs baseline: 2.2626x; 1.1532x over previous
import functools

import jax
import jax.numpy as jnp
from jax import lax
from jax.experimental import pallas as pl
from jax.experimental.pallas import tpu as pltpu
from jax.experimental.pallas import tpu_sc as plsc

EMB = 64
NC = 2
NS = 16
NW = NC * NS
CHUNK = 512


def _sc_gather_pair(table, idx_a, idx_b):
    n = idx_a.shape[0]
    per_w = n // NW
    n_chunks = per_w // CHUNK
    assert per_w % CHUNK == 0 and n % NW == 0

    mesh = plsc.VectorSubcoreMesh(core_axis_name="c", subcore_axis_name="s")

    @functools.partial(
        pl.kernel,
        out_type=(
            jax.ShapeDtypeStruct((n, EMB), jnp.float32),
            jax.ShapeDtypeStruct((n, EMB), jnp.float32),
        ),
        mesh=mesh,
        scratch_types=[
            pltpu.VMEM((per_w,), jnp.int32),
            pltpu.VMEM((per_w,), jnp.int32),
            pltpu.VMEM((CHUNK, EMB), jnp.float32),
            pltpu.VMEM((CHUNK, EMB), jnp.float32),
            pltpu.SemaphoreType.DMA,
            pltpu.SemaphoreType.DMA,
        ],
        compiler_params=pltpu.CompilerParams(use_tc_tiling_on_sc=False),
    )
    def gather_kernel(idx_a_hbm, idx_b_hbm, table_hbm, out_a_hbm, out_b_hbm,
                      idx_a_v, idx_b_v, buf0, buf1, sem0, sem1):
        wid = lax.axis_index("s") * NC + lax.axis_index("c")
        base = wid * per_w
        pltpu.sync_copy(idx_a_hbm.at[pl.ds(base, per_w)], idx_a_v)
        pltpu.sync_copy(idx_b_hbm.at[pl.ds(base, per_w)], idx_b_v)

        def one_array(idx_v, out_hbm):
            def body(c, _):
                off = c * CHUNK
                pltpu.async_copy(
                    table_hbm.at[idx_v.at[pl.ds(off, CHUNK)]], buf0, sem0
                ).wait()
                pltpu.sync_copy(buf0, out_hbm.at[pl.ds(base + off, CHUNK)])
                return ()

            lax.fori_loop(0, n_chunks, body, (), unroll=False)

        one_array(idx_a_v, out_a_hbm)
        one_array(idx_b_v, out_b_hbm)

    return gather_kernel(idx_a, idx_b, table)


def _tc_transpose_table(table):
    V = table.shape[0]
    VB = 8192
    K = VB // 2
    grid = (V + VB - 1) // VB

    def body(x_ref, y_ref):
        x = x_ref[...]
        xt_l = x[:, :K].T
        eye = (jax.lax.broadcasted_iota(jnp.int32, (EMB, EMB), 0)
               == jax.lax.broadcasted_iota(jnp.int32, (EMB, EMB), 1)
               ).astype(jnp.float32)
        xt_r = jax.lax.dot_general(
            x[:, K:], eye, (((0,), (0,)), ((), ())),
            preferred_element_type=jnp.float32,
        )
        y_ref[...] = jnp.concatenate([xt_l, xt_r], axis=1)

    return pl.pallas_call(
        body,
        grid=(grid,),
        in_specs=[pl.BlockSpec((EMB, VB), lambda j: (0, j))],
        out_specs=pl.BlockSpec((K, 2 * EMB), lambda j: (j, 0)),
        out_shape=jax.ShapeDtypeStruct((grid * K, 2 * EMB), jnp.float32),
    )(table.T)


def _tc_finalize(tmp_a, tmp_b, W, b, L, B):
    nb = B // 1024

    def body(ta_ref, tb_ref, w_ref, b_ref,
             ea1_ref, ea2_ref, eb1_ref, eb2_ref,
             pa1_ref, pa2_ref, pb1_ref, pb2_ref):
        w = w_ref[...]
        bc = b_ref[...]

        def one(t_ref, e1_ref, e2_ref, p1_ref, p2_ref):
            xt = t_ref[0, 0].T
            e = jnp.concatenate([xt[:EMB], xt[EMB:]], axis=1)
            e1_ref[0] = e
            e2_ref[0] = e
            p = jnp.dot(w, jnp.maximum(e, 0.0),
                        preferred_element_type=jnp.float32) + bc
            p1_ref[0] = p
            p2_ref[0] = p

        one(ta_ref, ea1_ref, ea2_ref, pa1_ref, pa2_ref)
        one(tb_ref, eb1_ref, eb2_ref, pb1_ref, pb2_ref)

    in_blk = pl.BlockSpec((1, 1, 512, 128), lambda l, j: (l, j, 0, 0))
    out_blk = pl.BlockSpec((1, EMB, 1024), lambda l, j: (l, 0, j))
    out_sh = jax.ShapeDtypeStruct((L, EMB, B), jnp.float32)
    return pl.pallas_call(
        body,
        grid=(L, nb),
        in_specs=[
            in_blk,
            in_blk,
            pl.BlockSpec((EMB, EMB), lambda l, j: (0, 0)),
            pl.BlockSpec((EMB, 1), lambda l, j: (0, 0)),
        ],
        out_specs=[out_blk] * 8,
        out_shape=(out_sh,) * 8,
    )(tmp_a, tmp_b, W, b)


def _permuted_flat_indices(t, L, B):
    v = t.astype(jnp.int32)
    r = v & (8192 - 1)
    i = (v & ~(8192 - 1)) + ((r & (4096 - 1)) << 1) + (r >> 12)
    tt = i.T.reshape(L, B // 1024, 2, 512)
    return tt.transpose(0, 1, 3, 2).reshape(L * B).astype(jnp.int32)


def kernel(tensor_day, tensor_diagnoses, table, W, b):
    B, L = tensor_day.shape
    n = B * L
    idx_diag = _permuted_flat_indices(tensor_diagnoses, L, B)
    idx_day = _permuted_flat_indices(tensor_day, L, B)

    table2 = _tc_transpose_table(table)
    table_lin = table2.reshape(table2.shape[0] * 2, EMB)

    tmp_diag, tmp_day = _sc_gather_pair(table_lin, idx_diag, idx_day)
    tmp_diag = tmp_diag.reshape(L, B // 1024, 512, 128)
    tmp_day = tmp_day.reshape(L, B // 1024, 512, 128)

    (ed_x, ed_y, ey_x, ey_y, pd_x, pd_y, py_x, py_y) = _tc_finalize(
        tmp_diag, tmp_day, W, b.reshape(EMB, 1), L, B
    )

    def to_logical(x):
        return jnp.transpose(x, (2, 0, 1))

    return (
        (to_logical(ed_x), to_logical(ey_x)),
        (to_logical(pd_x), to_logical(py_x)),
        (to_logical(ed_y), to_logical(ey_y)),
        (to_logical(pd_y), to_logical(py_y)),
    )
